# baseline (speedup 1.0000x reference)
import functools

import jax
import jax.numpy as jnp
from jax import lax
from jax.experimental import pallas as pl
from jax.experimental.pallas import tpu as pltpu
from jax.experimental.pallas import tpu_sc as plsc

VOCAB_ROWS = 1000002
WIDE_ROWS = 1000064
FULL_BLOCKS = 7813
DIM = 64
WIDE = 128
LANES = 16
NUM_CORES = 2
NUM_SUBCORES = 16
NUM_WORKERS = NUM_CORES * NUM_SUBCORES
CHUNK = 400
BLOCK_ITERS = 123


def _transpose_block(src_v, dst_v, ncols):
    @pl.loop(0, DIM, unroll=8)
    def _row(c):
        c_vec = jnp.full((LANES,), c, jnp.int32)
        for m in range(ncols // LANES):
            r_idx = lax.iota(jnp.int32, LANES) + m * LANES
            vals = src_v[c, pl.ds(m * LANES, LANES)]
            plsc.store_scatter(dst_v, [r_idx, c_vec], vals)


@jax.jit
def _format_table(table_t):
    mesh = plsc.VectorSubcoreMesh(core_axis_name="c", subcore_axis_name="s")

    @functools.partial(
        pl.kernel,
        out_type=jax.ShapeDtypeStruct((WIDE_ROWS, WIDE), jnp.float32),
        mesh=mesh,
        scratch_types=[
            pltpu.VMEM((DIM, 128), jnp.float32),
            pltpu.VMEM((DIM, 128), jnp.float32),
            pltpu.VMEM((128, WIDE), jnp.float32),
            pltpu.VMEM((128, WIDE), jnp.float32),
            pltpu.SemaphoreType.DMA,
            pltpu.SemaphoreType.DMA,
            pltpu.SemaphoreType.DMA,
            pltpu.SemaphoreType.DMA,
        ],
        compiler_params=pltpu.CompilerParams(
            needs_layout_passes=False, disable_bounds_checks=True),
    )
    def body(tab_hbm, wide_hbm, in0_v, in1_v, out0_v, out1_v,
             sem_i0, sem_i1, sem_o0, sem_o1):
        wid = lax.axis_index("s") * NUM_CORES + lax.axis_index("c")

        pltpu.async_copy(
            tab_hbm.at[:, pl.ds(wid * 128, 128)], in0_v, sem_i0)

        @pl.loop(0, BLOCK_ITERS)
        def _pair(j):
            blk0 = wid + NUM_WORKERS * 2 * j
            blk1 = blk0 + NUM_WORKERS

            @pl.when(blk1 < FULL_BLOCKS)
            def _():
                pltpu.async_copy(
                    tab_hbm.at[:, pl.ds(blk1 * 128, 128)], in1_v, sem_i1)

            @pl.when(blk0 < FULL_BLOCKS)
            def _():
                pltpu.make_async_copy(
                    tab_hbm.at[:, pl.ds(blk0 * 128, 128)], in0_v,
                    sem_i0).wait()

                @pl.when(j > 0)
                def _():
                    pltpu.make_async_copy(
                        out0_v, wide_hbm.at[pl.ds(blk0 * 128, 128)],
                        sem_o0).wait()

                _transpose_block(in0_v, out0_v, 128)
                pltpu.async_copy(
                    out0_v, wide_hbm.at[pl.ds(blk0 * 128, 128)], sem_o0)

                blk2 = blk1 + NUM_WORKERS

                @pl.when(blk2 < FULL_BLOCKS)
                def _():
                    pltpu.async_copy(
                        tab_hbm.at[:, pl.ds(blk2 * 128, 128)], in0_v, sem_i0)

            @pl.when(blk1 < FULL_BLOCKS)
            def _():
                pltpu.make_async_copy(
                    tab_hbm.at[:, pl.ds(blk1 * 128, 128)], in1_v,
                    sem_i1).wait()

                @pl.when(j > 0)
                def _():
                    pltpu.make_async_copy(
                        out1_v, wide_hbm.at[pl.ds(blk1 * 128, 128)],
                        sem_o1).wait()

                _transpose_block(in1_v, out1_v, 128)
                pltpu.async_copy(
                    out1_v, wide_hbm.at[pl.ds(blk1 * 128, 128)], sem_o1)

        pltpu.make_async_copy(
            out0_v, wide_hbm.at[pl.ds(0, 128)], sem_o0).wait()
        pltpu.make_async_copy(
            out1_v, wide_hbm.at[pl.ds(0, 128)], sem_o1).wait()

    return body(table_t)


@functools.partial(jax.jit, static_argnames=("total",))
def _gather_rows(ids_flat, table_wide, total):
    per_w = total // NUM_WORKERS
    n_chunks = per_w // CHUNK
    n_pairs = n_chunks // 2
    mesh = plsc.VectorSubcoreMesh(core_axis_name="c", subcore_axis_name="s")

    @functools.partial(
        pl.kernel,
        out_type=jax.ShapeDtypeStruct((total, WIDE), jnp.float32),
        mesh=mesh,
        scratch_types=[
            pltpu.VMEM((per_w,), jnp.int32),
            pltpu.VMEM((CHUNK, WIDE), jnp.float32),
            pltpu.VMEM((CHUNK, WIDE), jnp.float32),
            pltpu.SemaphoreType.DMA,
            pltpu.SemaphoreType.DMA,
        ],
    )
    def body(ids_hbm, table_hbm, out_hbm, idx_v, rows0_v, rows1_v, sem0, sem1):
        wid = lax.axis_index("s") * NUM_CORES + lax.axis_index("c")
        base = wid * per_w

        pltpu.sync_copy(ids_hbm.at[pl.ds(base, per_w)], idx_v)

        pltpu.async_copy(
            table_hbm.at[idx_v.at[pl.ds(0, CHUNK)]], rows0_v, sem0)

        @pl.loop(0, n_pairs)
        def _pair(j):
            i0 = 2 * j
            off0 = base + i0 * CHUNK
            off1 = off0 + CHUNK

            pltpu.async_copy(
                table_hbm.at[idx_v.at[pl.ds((i0 + 1) * CHUNK, CHUNK)]],
                rows1_v, sem1)

            pltpu.make_async_copy(
                table_hbm.at[idx_v.at[pl.ds(0, CHUNK)]], rows0_v, sem0).wait()
            pltpu.sync_copy(rows0_v, out_hbm.at[pl.ds(off0, CHUNK)])

            @pl.when(j < n_pairs - 1)
            def _():
                pltpu.async_copy(
                    table_hbm.at[idx_v.at[pl.ds((i0 + 2) * CHUNK, CHUNK)]],
                    rows0_v, sem0)

            pltpu.make_async_copy(
                table_hbm.at[idx_v.at[pl.ds(0, CHUNK)]], rows1_v, sem1).wait()
            pltpu.sync_copy(rows1_v, out_hbm.at[pl.ds(off1, CHUNK)])

    return body(ids_flat, table_wide)


def kernel(ids, pads, table):
    B, L = ids.shape
    total = B * L
    table_wide = _format_table(table.T)
    rows = _gather_rows(ids.reshape(total), table_wide, total)
    return rows[:, :DIM].reshape(B, L, DIM), pads


# speedup vs baseline: 1.6693x; 1.6693x over previous
import functools

import jax
import jax.numpy as jnp
from jax import lax
from jax.experimental import pallas as pl
from jax.experimental.pallas import tpu as pltpu
from jax.experimental.pallas import tpu_sc as plsc
from jax.experimental.layout import Layout, with_layout_constraint

DIM = 64
WIDE = 128
NUM_CORES = 2
NUM_SUBCORES = 16
NUM_WORKERS = NUM_CORES * NUM_SUBCORES
CHUNK = 400


@functools.partial(jax.jit, static_argnames=("total",))
def _gather_rows(ids_flat, table_wide, total):
    per_w = total // NUM_WORKERS
    n_chunks = per_w // CHUNK
    n_pairs = n_chunks // 2
    mesh = plsc.VectorSubcoreMesh(core_axis_name="c", subcore_axis_name="s")

    @functools.partial(
        pl.kernel,
        out_type=jax.ShapeDtypeStruct((total, WIDE), jnp.float32),
        mesh=mesh,
        scratch_types=[
            pltpu.VMEM((per_w,), jnp.int32),
            pltpu.VMEM((CHUNK, WIDE), jnp.float32),
            pltpu.VMEM((CHUNK, WIDE), jnp.float32),
            pltpu.SemaphoreType.DMA,
            pltpu.SemaphoreType.DMA,
        ],
    )
    def body(ids_hbm, table_hbm, out_hbm, idx_v, rows0_v, rows1_v, sem0, sem1):
        wid = lax.axis_index("s") * NUM_CORES + lax.axis_index("c")
        base = wid * per_w

        pltpu.sync_copy(ids_hbm.at[pl.ds(base, per_w)], idx_v)

        pltpu.async_copy(
            table_hbm.at[idx_v.at[pl.ds(0, CHUNK)]], rows0_v, sem0)

        @pl.loop(0, n_pairs)
        def _pair(j):
            i0 = 2 * j
            off0 = base + i0 * CHUNK
            off1 = off0 + CHUNK

            pltpu.async_copy(
                table_hbm.at[idx_v.at[pl.ds((i0 + 1) * CHUNK, CHUNK)]],
                rows1_v, sem1)

            pltpu.make_async_copy(
                table_hbm.at[idx_v.at[pl.ds(0, CHUNK)]], rows0_v, sem0).wait()
            pltpu.sync_copy(rows0_v, out_hbm.at[pl.ds(off0, CHUNK)])

            @pl.when(j < n_pairs - 1)
            def _():
                pltpu.async_copy(
                    table_hbm.at[idx_v.at[pl.ds((i0 + 2) * CHUNK, CHUNK)]],
                    rows0_v, sem0)

            pltpu.make_async_copy(
                table_hbm.at[idx_v.at[pl.ds(0, CHUNK)]], rows1_v, sem1).wait()
            pltpu.sync_copy(rows1_v, out_hbm.at[pl.ds(off1, CHUNK)])

    return body(ids_flat, table_wide)


def kernel(ids, pads, table):
    B, L = ids.shape
    total = B * L
    table_rm = with_layout_constraint(
        table, Layout(major_to_minor=(0, 1), tiling=((8, 128),)))
    table_wide = jnp.pad(table_rm, ((0, 0), (0, WIDE - DIM)))
    rows = _gather_rows(ids.reshape(total), table_wide, total)
    return rows[:, :DIM].reshape(B, L, DIM), pads
